# Initial kernel scaffold; baseline (speedup 1.0000x reference)
import jax
import jax.numpy as jnp
from jax.experimental import pallas as pl


def kernel(dispatched_input, tokens_per_expert, W1, W2):
    raise NotImplementedError("write your pallas kernel here")


# speedup vs baseline: 1.0571x; 1.0571x over previous
import jax
import jax.numpy as jnp
from jax.experimental import pallas as pl
from jax.experimental.pallas import tpu as pltpu


def _expert_mlp_kernel(x_ref, w1_ref, w2_ref, y_ref):
    x = x_ref[...]
    h = jnp.dot(x, w1_ref[0], preferred_element_type=jnp.float32)
    inner = 0.7978845608028654 * (h + 0.044715 * (h * h * h))
    g = 0.5 * h * (1.0 + jnp.tanh(inner))
    y_ref[...] = jnp.dot(g, w2_ref[0], preferred_element_type=jnp.float32)


def kernel(dispatched_input, tokens_per_expert, W1, W2):
    E, D, F = W1.shape
    cap = dispatched_input.shape[0] // E
    out = pl.pallas_call(
        _expert_mlp_kernel,
        grid=(E,),
        in_specs=[
            pl.BlockSpec((cap, D), lambda e: (e, 0)),
            pl.BlockSpec((1, D, F), lambda e: (e, 0, 0)),
            pl.BlockSpec((1, F, D), lambda e: (e, 0, 0)),
        ],
        out_specs=pl.BlockSpec((cap, D), lambda e: (e, 0)),
        out_shape=jax.ShapeDtypeStruct((E * cap, D), jnp.float32),
        compiler_params=pltpu.CompilerParams(
            dimension_semantics=("arbitrary",),
            vmem_limit_bytes=60 * 1024 * 1024,
        ),
    )(dispatched_input, W1, W2)
    return out
